# Initial kernel scaffold; baseline (speedup 1.0000x reference)
import jax
import jax.numpy as jnp
from jax.experimental import pallas as pl


def kernel(inputs, token_table, position_table):
    raise NotImplementedError("write your pallas kernel here")


# speedup vs baseline: 4.2575x; 4.2575x over previous
import functools

import jax
import jax.numpy as jnp
from jax import lax
from jax.experimental import pallas as pl
from jax.experimental.pallas import tpu as pltpu
from jax.experimental.pallas import tpu_sc as plsc

SEQ = 200
EMBED = 128
BATCH = 1024
NW = 32
SEQ_PER_W = BATCH // NW
HALF = SEQ // 2
LANES = 16


def _sc_body(idx_hbm, table_hbm, pos_hbm, out_hbm, idx_v, pos_v, rows_v, gsem):
    wid = lax.axis_index("s") * 2 + lax.axis_index("c")
    pltpu.sync_copy(idx_hbm.at[wid], idx_v)
    pltpu.sync_copy(pos_hbm, pos_v)

    def seq_body(seq, carry):
        cp0 = pltpu.async_copy(
            table_hbm.at[idx_v.at[2 * seq]], rows_v.at[pl.ds(0, HALF)], gsem)
        cp1 = pltpu.async_copy(
            table_hbm.at[idx_v.at[2 * seq + 1]], rows_v.at[pl.ds(HALF, HALF)], gsem)
        cp0.wait()
        cp1.wait()

        def add_body(r, c):
            for j in range(EMBED // LANES):
                sl = pl.ds(j * LANES, LANES)
                plsc.addupdate(rows_v.at[r, sl], pos_v[r, sl])
            return c

        lax.fori_loop(0, SEQ, add_body, 0)
        pltpu.sync_copy(rows_v, out_hbm.at[wid * SEQ_PER_W + seq])
        return carry

    lax.fori_loop(0, SEQ_PER_W, seq_body, 0)


@functools.partial(jax.jit, static_argnums=())
def _run(idx3, token_table, position_table):
    mesh = plsc.VectorSubcoreMesh(core_axis_name="c", subcore_axis_name="s")
    fn = functools.partial(
        pl.kernel,
        out_type=jax.ShapeDtypeStruct((BATCH, SEQ, EMBED), jnp.float32),
        mesh=mesh,
        scratch_types=[
            pltpu.VMEM((2 * SEQ_PER_W, HALF), jnp.int32),
            pltpu.VMEM((SEQ, EMBED), jnp.float32),
            pltpu.VMEM((SEQ, EMBED), jnp.float32),
            pltpu.SemaphoreType.DMA,
        ],
    )(_sc_body)
    return fn(idx3, token_table, position_table)


def kernel(inputs, token_table, position_table):
    idx3 = inputs.astype(jnp.int32).reshape(NW, 2 * SEQ_PER_W, HALF)
    return _run(idx3, token_table, position_table)


# speedup vs baseline: 7.2721x; 1.7081x over previous
import functools

import jax
import jax.numpy as jnp
from jax import lax
from jax.experimental import pallas as pl
from jax.experimental.pallas import tpu as pltpu
from jax.experimental.pallas import tpu_sc as plsc

SEQ = 200
EMBED = 128
BATCH = 1024
NW = 32
SEQ_PER_W = BATCH // NW
HALF = SEQ // 2
LANES = 16
NBUF = 3


def _sc_body(idx_hbm, table_hbm, pos_hbm, out_hbm,
             idx_v, pos_v, r0, r1, r2, g0, g1, g2, o0, o1, o2):
    rows = [r0, r1, r2]
    gsem = [g0, g1, g2]
    osem = [o0, o1, o2]
    wid = lax.axis_index("s") * 2 + lax.axis_index("c")
    pltpu.sync_copy(idx_hbm.at[wid], idx_v)
    pltpu.sync_copy(pos_hbm, pos_v)

    def start_gather(s, b):
        cp0 = pltpu.async_copy(
            table_hbm.at[idx_v.at[2 * s]], rows[b].at[pl.ds(0, HALF)], gsem[b])
        cp1 = pltpu.async_copy(
            table_hbm.at[idx_v.at[2 * s + 1]], rows[b].at[pl.ds(HALF, HALF)], gsem[b])
        return (cp0, cp1)

    def add_pos(b):
        def add_body(r, c):
            for j in range(EMBED // LANES):
                sl = pl.ds(j * LANES, LANES)
                plsc.addupdate(rows[b].at[r, sl], pos_v[r, sl])
            return c
        lax.fori_loop(0, SEQ, add_body, 0)

    pend_g = {0: start_gather(0, 0), 1: start_gather(1, 1)}
    pend_o = {}
    for s in range(SEQ_PER_W):
        b = s % NBUF
        for cp in pend_g.pop(s):
            cp.wait()
        add_pos(b)
        pend_o[s] = pltpu.async_copy(
            rows[b], out_hbm.at[wid * SEQ_PER_W + s], osem[b])
        if s - 1 in pend_o:
            pend_o.pop(s - 1).wait()
        if s + 2 < SEQ_PER_W:
            pend_g[s + 2] = start_gather(s + 2, (s + 2) % NBUF)
    for s in sorted(pend_o):
        pend_o.pop(s).wait()


@jax.jit
def _run(idx3, token_table, position_table):
    mesh = plsc.VectorSubcoreMesh(core_axis_name="c", subcore_axis_name="s")
    fn = functools.partial(
        pl.kernel,
        out_type=jax.ShapeDtypeStruct((BATCH, SEQ, EMBED), jnp.float32),
        mesh=mesh,
        scratch_types=[
            pltpu.VMEM((2 * SEQ_PER_W, HALF), jnp.int32),
            pltpu.VMEM((SEQ, EMBED), jnp.float32),
            pltpu.VMEM((SEQ, EMBED), jnp.float32),
            pltpu.VMEM((SEQ, EMBED), jnp.float32),
            pltpu.VMEM((SEQ, EMBED), jnp.float32),
            pltpu.SemaphoreType.DMA,
            pltpu.SemaphoreType.DMA,
            pltpu.SemaphoreType.DMA,
            pltpu.SemaphoreType.DMA,
            pltpu.SemaphoreType.DMA,
            pltpu.SemaphoreType.DMA,
        ],
    )(_sc_body)
    return fn(idx3, token_table, position_table)


def kernel(inputs, token_table, position_table):
    idx3 = inputs.astype(jnp.int32).reshape(NW, 2 * SEQ_PER_W, HALF)
    return _run(idx3, token_table, position_table)
